# baseline (speedup 1.0000x reference)
import functools

import jax
import jax.numpy as jnp
from jax import lax
from jax.experimental import pallas as pl
from jax.experimental.pallas import tpu as pltpu
from jax.experimental.pallas import tpu_sc as plsc

N = 1024
D = 128
KMEM = 100000
SL = 32
BAND = 2
RHO = 0.5
CHUNK = 128
TK = 4096
NT = (KMEM + TK - 1) // TK
KPAD = NT * TK
NCHUNK = KPAD // CHUNK
NCAND = SL * CHUNK
NEG = -1e30
_PREC = lax.Precision.HIGHEST


def _sc_gather(table, idx):
    B = idx.shape[0]
    d = table.shape[1]
    info = plsc.get_sparse_core_info()
    nc = info.num_cores
    nw = nc * info.num_subcores
    b_per_w = B // nw
    rchunk = min(b_per_w, 128)
    nloops = b_per_w // rchunk
    mesh = plsc.VectorSubcoreMesh(core_axis_name="c", subcore_axis_name="s")

    @functools.partial(
        pl.kernel, mesh=mesh,
        out_type=jax.ShapeDtypeStruct((B, d), jnp.float32),
        scratch_types=[
            pltpu.VMEM((b_per_w,), jnp.int32),
            pltpu.VMEM((rchunk, d), jnp.float32),
            pltpu.SemaphoreType.DMA,
        ],
    )
    def k(table_hbm, idx_hbm, out_hbm, idx_v, rows_v, sem):
        wid = lax.axis_index("s") * nc + lax.axis_index("c")
        base = wid * b_per_w
        pltpu.sync_copy(idx_hbm.at[pl.ds(base, b_per_w)], idx_v)
        for c in range(nloops):
            src = idx_v if nloops == 1 else idx_v.at[pl.ds(c * rchunk, rchunk)]
            pltpu.async_copy(table_hbm.at[src], rows_v, sem).wait()
            pltpu.sync_copy(rows_v, out_hbm.at[pl.ds(base + c * rchunk, rchunk)])

    return k(table, idx)


def _sim_chunkmax(E, M):
    def body(e_ref, m_ref, sim_ref, cm_ref, x_ref):
        j = pl.program_id(0)
        s = lax.dot_general(e_ref[...], m_ref[...], (((1,), (1,)), ((), ())),
                            preferred_element_type=jnp.float32,
                            precision=lax.Precision.DEFAULT)
        col = j * TK + lax.broadcasted_iota(jnp.int32, (N, TK), 1)
        s = jnp.where(col < KMEM, s, NEG)
        sim_ref[...] = s
        cm_ref[0] = jnp.max(s.reshape(N, TK // CHUNK, CHUNK), axis=2)

        @pl.when(j == 0)
        def _():
            r = lax.broadcasted_iota(jnp.int32, (N, N), 0)
            c = lax.broadcasted_iota(jnp.int32, (N, N), 1)
            tril = (r >= c).astype(jnp.float32)
            cs = lax.dot_general(tril, e_ref[...], (((1,), (0,)), ((), ())),
                                 preferred_element_type=jnp.float32,
                                 precision=_PREC)
            denom = (lax.broadcasted_iota(jnp.int32, (N, 1), 0)
                     .astype(jnp.float32) + 1.0)
            x_ref[...] = cs / denom

    sim, cm3, X = pl.pallas_call(
        body,
        grid=(NT,),
        in_specs=[pl.BlockSpec((N, D), lambda j: (0, 0)),
                  pl.BlockSpec((TK, D), lambda j: (j, 0))],
        out_specs=[pl.BlockSpec((N, TK), lambda j: (0, j)),
                   pl.BlockSpec((1, N, TK // CHUNK), lambda j: (j, 0, 0)),
                   pl.BlockSpec((N, D), lambda j: (0, 0))],
        out_shape=[jax.ShapeDtypeStruct((N, KPAD), jnp.float32),
                   jax.ShapeDtypeStruct((NT, N, TK // CHUNK), jnp.float32),
                   jax.ShapeDtypeStruct((N, D), jnp.float32)],
    )(E, M)
    return sim, cm3.transpose(1, 0, 2).reshape(N, NCHUNK), X


def _top_chunks(cm):
    def body(cm_ref, cid_ref, gid_ref):
        C = cm_ref[...]
        ids = lax.broadcasted_iota(jnp.int32, (N, NCHUNK), 1)
        row = lax.broadcasted_iota(jnp.int32, (N, 1), 0)
        big = jnp.int32(2 ** 30)
        for k in range(SL):
            m = jnp.max(C, axis=1, keepdims=True)
            pick = jnp.min(jnp.where(C >= m, ids, big), axis=1, keepdims=True)
            cid_ref[:, k:k + 1] = pick
            gid_ref[:, k:k + 1] = row * NCHUNK + pick
            C = jnp.where(ids == pick, NEG, C)

    return pl.pallas_call(
        body,
        out_shape=[jax.ShapeDtypeStruct((N, SL), jnp.int32),
                   jax.ShapeDtypeStruct((N, SL), jnp.int32)],
    )(cm)


def _final_topk(cand, cids):
    def body(v_ref, c_ref, ks_ref):
        V = v_ref[...]
        cid = c_ref[...]
        ids = lax.broadcasted_iota(jnp.int32, (N, NCAND), 1)
        ids_sl = lax.broadcasted_iota(jnp.int32, (N, SL), 1)
        big = jnp.int32(2 ** 30)
        for k in range(SL):
            m = jnp.max(V, axis=1, keepdims=True)
            pick = jnp.min(jnp.where(V >= m, ids, big), axis=1, keepdims=True)
            V = jnp.where(ids == pick, NEG, V)
            c_star = jnp.right_shift(pick, 7)
            r = jnp.bitwise_and(pick, jnp.int32(CHUNK - 1))
            sel = jnp.max(jnp.where(ids_sl == c_star, cid, jnp.int32(-1)),
                          axis=1, keepdims=True)
            ks_ref[:, k:k + 1] = sel * CHUNK + r

    return pl.pallas_call(
        body,
        out_shape=jax.ShapeDtypeStruct((N, SL), jnp.int32),
    )(cand, cids)


def _shift(Y, o):
    if o == 0:
        return Y
    z = jnp.zeros((abs(o), D), jnp.float32)
    if o > 0:
        return jnp.concatenate([Y[o:, :], z], axis=0)
    return jnp.concatenate([z, Y[:N + o, :]], axis=0)


def _pdhg_step(P, Lam, X, Msub, wpad):
    def body(p_ref, l_ref, x_ref, ms_ref, w_ref, po_ref, lo_ref):
        wv = w_ref[...]
        wn = wv / (jnp.sum(jnp.abs(wv)) + 1e-6)
        Ms = ms_ref[...]
        P = p_ref[...]
        Yc = jnp.sum(P[:, :, None] * Ms, axis=1)
        bf = jnp.zeros((N, D), jnp.float32)
        for j in range(2 * BAND + 1):
            bf = bf + wn[0, j] * _shift(Yc, j - BAND)
        R = bf - x_ref[...]
        Lam2 = l_ref[...] + RHO * R
        g = Lam2 + R
        gp = jnp.sum(Ms * g[:, None, :], axis=2)
        logP = jnp.log(P + 1e-9) - RHO * gp
        mx = jnp.max(logP, axis=1, keepdims=True)
        e = jnp.exp(logP - mx)
        po_ref[...] = e / jnp.sum(e, axis=1, keepdims=True)
        lo_ref[...] = Lam2

    return pl.pallas_call(
        body,
        out_shape=[jax.ShapeDtypeStruct((N, SL), jnp.float32),
                   jax.ShapeDtypeStruct((N, D), jnp.float32)],
    )(P, Lam, X, Msub, wpad)


def _mkY(P, Msub):
    def body(p_ref, ms_ref, y_ref):
        y_ref[...] = jnp.sum(p_ref[...][:, :, None] * ms_ref[...], axis=1)

    return pl.pallas_call(
        body,
        out_shape=jax.ShapeDtypeStruct((N, D), jnp.float32),
    )(P, Msub)


def _readout(Y, M):
    def body(y_ref, m_ref, o_ref):
        o_ref[...] = lax.dot_general(y_ref[...], m_ref[...],
                                     (((1,), (1,)), ((), ())),
                                     preferred_element_type=jnp.float32,
                                     precision=lax.Precision.DEFAULT)

    return pl.pallas_call(
        body,
        grid=(NT,),
        in_specs=[pl.BlockSpec((N, D), lambda j: (0, 0)),
                  pl.BlockSpec((TK, D), lambda j: (j, 0))],
        out_specs=pl.BlockSpec((N, TK), lambda j: (0, j)),
        out_shape=jax.ShapeDtypeStruct((N, KMEM), jnp.float32),
    )(Y, M)


def kernel(tokens, embed, M, w, T):
    tokens = tokens.astype(jnp.int32)
    E = _sc_gather(embed, tokens)
    sim, cm, X = _sim_chunkmax(E, M)
    cids, gids = _top_chunks(cm)
    cand = _sc_gather(sim.reshape(N * NCHUNK, CHUNK),
                      gids.reshape(-1)).reshape(N, NCAND)
    kset = _final_topk(cand, cids)
    msub = _sc_gather(M, kset.reshape(-1)).reshape(N, SL, D)
    wpad = jnp.pad(w.astype(jnp.float32), (0, 3)).reshape(1, 8)
    P0 = jnp.full((N, SL), 1.0 / SL, jnp.float32)
    Lam0 = jnp.zeros((N, D), jnp.float32)

    def step(_, carry):
        P, Lam = carry
        return _pdhg_step(P, Lam, X, msub, wpad)

    P, _ = lax.fori_loop(0, T, step, (P0, Lam0))
    return _readout(_mkY(P, msub), M)


# speedup vs baseline: 1.0299x; 1.0299x over previous
import functools

import jax
import jax.numpy as jnp
from jax import lax
from jax.experimental import pallas as pl
from jax.experimental.pallas import tpu as pltpu
from jax.experimental.pallas import tpu_sc as plsc

N = 1024
D = 128
KMEM = 100000
SL = 32
BAND = 2
RHO = 0.5
CHUNK = 128
TK = 2048
NT = (KMEM + TK - 1) // TK
KPAD = NT * TK
NCHUNK = KPAD // CHUNK
NCAND = SL * CHUNK
NEG = -1e30
_PREC = lax.Precision.HIGHEST


def _sc_gather(table, idx):
    B = idx.shape[0]
    d = table.shape[1]
    info = plsc.get_sparse_core_info()
    nc = info.num_cores
    nw = nc * info.num_subcores
    b_per_w = B // nw
    rchunk = min(b_per_w, 128)
    nloops = b_per_w // rchunk
    mesh = plsc.VectorSubcoreMesh(core_axis_name="c", subcore_axis_name="s")

    @functools.partial(
        pl.kernel, mesh=mesh,
        out_type=jax.ShapeDtypeStruct((B, d), jnp.float32),
        scratch_types=[
            pltpu.VMEM((b_per_w,), jnp.int32),
            pltpu.VMEM((rchunk, d), jnp.float32),
            pltpu.SemaphoreType.DMA,
        ],
    )
    def k(table_hbm, idx_hbm, out_hbm, idx_v, rows_v, sem):
        wid = lax.axis_index("s") * nc + lax.axis_index("c")
        base = wid * b_per_w
        pltpu.sync_copy(idx_hbm.at[pl.ds(base, b_per_w)], idx_v)
        for c in range(nloops):
            src = idx_v if nloops == 1 else idx_v.at[pl.ds(c * rchunk, rchunk)]
            pltpu.async_copy(table_hbm.at[src], rows_v, sem).wait()
            pltpu.sync_copy(rows_v, out_hbm.at[pl.ds(base + c * rchunk, rchunk)])

    return k(table, idx)


def _sim_chunkmax(E, M):
    def body(e_ref, m_ref, sim_ref, cm_ref):
        j = pl.program_id(0)
        s = lax.dot_general(e_ref[...], m_ref[...], (((1,), (1,)), ((), ())),
                            preferred_element_type=jnp.float32,
                            precision=lax.Precision.DEFAULT)
        col = j * TK + lax.broadcasted_iota(jnp.int32, (N, TK), 1)
        s = jnp.where(col < KMEM, s, NEG)
        sim_ref[...] = s
        cm_ref[0] = jnp.max(s.reshape(N, TK // CHUNK, CHUNK), axis=2)

    sim, cm3 = pl.pallas_call(
        body,
        grid=(NT,),
        in_specs=[pl.BlockSpec((N, D), lambda j: (0, 0)),
                  pl.BlockSpec((TK, D), lambda j: (j, 0))],
        out_specs=[pl.BlockSpec((N, TK), lambda j: (0, j)),
                   pl.BlockSpec((1, N, TK // CHUNK), lambda j: (j, 0, 0))],
        out_shape=[jax.ShapeDtypeStruct((N, KPAD), jnp.float32),
                   jax.ShapeDtypeStruct((NT, N, TK // CHUNK), jnp.float32)],
    )(E, M)
    return sim, cm3.transpose(1, 0, 2).reshape(N, NCHUNK)


def _top_chunks(cm):
    def body(cm_ref, cid_ref, gid_ref):
        C = cm_ref[...]
        ids = lax.broadcasted_iota(jnp.int32, (N, NCHUNK), 1)
        row = lax.broadcasted_iota(jnp.int32, (N, 1), 0)
        big = jnp.int32(2 ** 30)
        for k in range(SL):
            m = jnp.max(C, axis=1, keepdims=True)
            pick = jnp.min(jnp.where(C >= m, ids, big), axis=1, keepdims=True)
            cid_ref[:, k:k + 1] = pick
            gid_ref[:, k:k + 1] = row * NCHUNK + pick
            C = jnp.where(ids == pick, NEG, C)

    return pl.pallas_call(
        body,
        out_shape=[jax.ShapeDtypeStruct((N, SL), jnp.int32),
                   jax.ShapeDtypeStruct((N, SL), jnp.int32)],
    )(cm)


def _final_topk(cand, cids):
    def body(v_ref, c_ref, ks_ref):
        V = v_ref[...]
        cid = c_ref[...]
        ids = lax.broadcasted_iota(jnp.int32, (N, NCAND), 1)
        ids_sl = lax.broadcasted_iota(jnp.int32, (N, SL), 1)
        big = jnp.int32(2 ** 30)
        for k in range(SL):
            m = jnp.max(V, axis=1, keepdims=True)
            pick = jnp.min(jnp.where(V >= m, ids, big), axis=1, keepdims=True)
            V = jnp.where(ids == pick, NEG, V)
            c_star = jnp.right_shift(pick, 7)
            r = jnp.bitwise_and(pick, jnp.int32(CHUNK - 1))
            sel = jnp.max(jnp.where(ids_sl == c_star, cid, jnp.int32(-1)),
                          axis=1, keepdims=True)
            ks_ref[:, k:k + 1] = sel * CHUNK + r

    return pl.pallas_call(
        body,
        out_shape=jax.ShapeDtypeStruct((N, SL), jnp.int32),
    )(cand, cids)


def _init(E, Msub):
    def body(e_ref, ms_ref, x_ref, y_ref):
        r = lax.broadcasted_iota(jnp.int32, (N, N), 0)
        c = lax.broadcasted_iota(jnp.int32, (N, N), 1)
        tril = (r >= c).astype(jnp.float32)
        cs = lax.dot_general(tril, e_ref[...], (((1,), (0,)), ((), ())),
                             preferred_element_type=jnp.float32,
                             precision=_PREC)
        denom = lax.broadcasted_iota(jnp.int32, (N, 1), 0).astype(jnp.float32) + 1.0
        x_ref[...] = cs / denom
        y_ref[...] = jnp.mean(ms_ref[...], axis=1)

    return pl.pallas_call(
        body,
        out_shape=[jax.ShapeDtypeStruct((N, D), jnp.float32),
                   jax.ShapeDtypeStruct((N, D), jnp.float32)],
    )(E, Msub)


def _shift(Y, o):
    if o == 0:
        return Y
    z = jnp.zeros((abs(o), D), jnp.float32)
    if o > 0:
        return jnp.concatenate([Y[o:, :], z], axis=0)
    return jnp.concatenate([z, Y[:N + o, :]], axis=0)


def _pdhg_step(P, Y, Lam, X, Msub, wpad):
    def body(p_ref, y_ref, l_ref, x_ref, ms_ref, w_ref,
             po_ref, yo_ref, lo_ref):
        wv = w_ref[...]
        wn = wv / (jnp.sum(jnp.abs(wv)) + 1e-6)
        Yc = y_ref[...]
        bf = jnp.zeros((N, D), jnp.float32)
        for j in range(2 * BAND + 1):
            bf = bf + wn[0, j] * _shift(Yc, j - BAND)
        R = bf - x_ref[...]
        Lam2 = l_ref[...] + RHO * R
        g = Lam2 + R
        Ms = ms_ref[...]
        gp = jnp.sum(Ms * g[:, None, :], axis=2)
        logP = jnp.log(p_ref[...] + 1e-9) - RHO * gp
        mx = jnp.max(logP, axis=1, keepdims=True)
        e = jnp.exp(logP - mx)
        Pn = e / jnp.sum(e, axis=1, keepdims=True)
        po_ref[...] = Pn
        yo_ref[...] = jnp.sum(Pn[:, :, None] * Ms, axis=1)
        lo_ref[...] = Lam2

    return pl.pallas_call(
        body,
        out_shape=[jax.ShapeDtypeStruct((N, SL), jnp.float32),
                   jax.ShapeDtypeStruct((N, D), jnp.float32),
                   jax.ShapeDtypeStruct((N, D), jnp.float32)],
    )(P, Y, Lam, X, Msub, wpad)


def _readout(Y, M):
    def body(y_ref, m_ref, o_ref):
        o_ref[...] = lax.dot_general(y_ref[...], m_ref[...],
                                     (((1,), (1,)), ((), ())),
                                     preferred_element_type=jnp.float32,
                                     precision=lax.Precision.DEFAULT)

    return pl.pallas_call(
        body,
        grid=(NT,),
        in_specs=[pl.BlockSpec((N, D), lambda j: (0, 0)),
                  pl.BlockSpec((TK, D), lambda j: (j, 0))],
        out_specs=pl.BlockSpec((N, TK), lambda j: (0, j)),
        out_shape=jax.ShapeDtypeStruct((N, KMEM), jnp.float32),
    )(Y, M)


def kernel(tokens, embed, M, w, T):
    tokens = tokens.astype(jnp.int32)
    E = _sc_gather(embed, tokens)
    sim, cm = _sim_chunkmax(E, M)
    cids, gids = _top_chunks(cm)
    cand = _sc_gather(sim.reshape(N * NCHUNK, CHUNK),
                      gids.reshape(-1)).reshape(N, NCAND)
    kset = _final_topk(cand, cids)
    msub = _sc_gather(M, kset.reshape(-1)).reshape(N, SL, D)
    X, Y0 = _init(E, msub)
    wpad = jnp.pad(w.astype(jnp.float32), (0, 3)).reshape(1, 8)
    P0 = jnp.full((N, SL), 1.0 / SL, jnp.float32)
    Lam0 = jnp.zeros((N, D), jnp.float32)

    def step(_, carry):
        P, Y, Lam = carry
        return _pdhg_step(P, Y, Lam, X, msub, wpad)

    _, Y, _ = lax.fori_loop(0, T, step, (P0, Y0, Lam0))
    return _readout(Y, M)
